# Initial kernel scaffold; baseline (speedup 1.0000x reference)
import jax
import jax.numpy as jnp
from jax.experimental import pallas as pl


def kernel(species, coordinates, net_charge, params):
    raise NotImplementedError("write your pallas kernel here")


# speedup vs baseline: 1.3659x; 1.3659x over previous
import jax
import jax.numpy as jnp
from jax import lax
from jax.experimental import pallas as pl
from jax.experimental.pallas import tpu as pltpu

A0 = 0.529177249
RC = 5.2
AEV_LEN = 512
N_SPECIES = 4


def _dot3(x, w):
    xh = x.astype(jnp.bfloat16)
    xl = (x - xh.astype(jnp.float32)).astype(jnp.bfloat16)
    wh = w.astype(jnp.bfloat16)
    wl = (w - wh.astype(jnp.float32)).astype(jnp.bfloat16)
    f32 = jnp.float32
    return (jnp.dot(xh, wh, preferred_element_type=f32)
            + jnp.dot(xh, wl, preferred_element_type=f32)
            + jnp.dot(xl, wh, preferred_element_type=f32))


def _celu(x):
    return jnp.where(x > 0, x, 0.1 * (jnp.exp(jnp.minimum(x, 0.0) / 0.1) - 1.0))


def _fused_kernel(spf_ref, coordsf_ref, cx_ref, cy_ref, cz_ref, nc_ref,
                  wa_ref, benv_ref,
                  qw1_ref, qb1_ref, qw2_ref, qb2_ref, qw3_ref, qb3_ref,
                  qw4_ref, qb4_ref,
                  aw1_ref, awq_ref, awe_ref, ab1_ref,
                  aw2_ref, ab2_ref, aw3_ref, ab3_ref, aw4_ref, ab4_ref,
                  energy_ref, pred_ref):
    G, n = cx_ref.shape
    R = G * n
    coordsf = coordsf_ref[...]
    cxs = coordsf[:, 0:1].reshape(G, n, 1)
    cys = coordsf[:, 1:2].reshape(G, n, 1)
    czs = coordsf[:, 2:3].reshape(G, n, 1)
    cxl = cx_ref[...][:, None, :]
    cyl = cy_ref[...][:, None, :]
    czl = cz_ref[...][:, None, :]

    dx = cxs - cxl
    dy = cys - cyl
    dz = czs - czl
    d2 = dx * dx + dy * dy + dz * dz
    row = lax.broadcasted_iota(jnp.int32, (n, n), 0)
    col = lax.broadcasted_iota(jnp.int32, (n, n), 1)
    offdiag = (row != col).astype(jnp.float32)[None]

    d_aev = jnp.sqrt(d2 + 1e-12)
    fc = 0.5 * jnp.cos(jnp.pi * jnp.minimum(d_aev, RC) / RC) + 0.5
    env = jnp.sum(fc * offdiag, axis=-1, keepdims=True)
    def _b(v):
        return v.astype(jnp.bfloat16).astype(jnp.float32)

    lin = (_b(cxs.reshape(R, 1)) * _b(wa_ref[0:1, :])
           + _b(cys.reshape(R, 1)) * _b(wa_ref[1:2, :])
           + _b(czs.reshape(R, 1)) * _b(wa_ref[2:3, :]))
    aev = jnp.tanh(lin + env.reshape(R, 1) * benv_ref[...])

    f32 = jnp.float32
    h = _celu(jnp.dot(aev, qw1_ref[...], preferred_element_type=f32) + qb1_ref[...])
    h = _celu(jnp.dot(h, qw2_ref[...], preferred_element_type=f32) + qb2_ref[...])
    h = _celu(jnp.dot(h, qw3_ref[...], preferred_element_type=f32) + qb3_ref[...])
    q = jnp.sum(h * qw4_ref[...], axis=-1, keepdims=True) + qb4_ref[...]

    q3 = q.reshape(G, n, 1)
    denom = jnp.maximum(jnp.sum(q3 * q3, axis=1, keepdims=True), 1e-8)
    excess = nc_ref[...] - jnp.sum(q3, axis=1, keepdims=True)
    pred = q3 + excess * (q3 * q3) / denom

    dist = jnp.sqrt(d2 + 1e-16) * (1.0 / A0)
    rinv = offdiag / dist
    esp = lax.dot_general(rinv, pred, (((2,), (1,)), ((0,), (0,))),
                          preferred_element_type=jnp.float32)
    coulomb = 0.5 * jnp.sum(pred * esp, axis=1, keepdims=True)

    predf = pred.reshape(R, 1)
    espf = esp.reshape(R, 1)
    spf = spf_ref[...]
    out = jnp.zeros((R, 1), jnp.float32)
    for s in range(N_SPECIES):
        hs = _celu(jnp.dot(aev, aw1_ref[s], preferred_element_type=jnp.float32)
                   + predf * awq_ref[s] + espf * awe_ref[s] + ab1_ref[s])
        hs = _celu(jnp.dot(hs, aw2_ref[s], preferred_element_type=jnp.float32)
                   + ab2_ref[s])
        hs = _celu(jnp.dot(hs, aw3_ref[s], preferred_element_type=jnp.float32)
                   + ab3_ref[s])
        o = jnp.sum(hs * aw4_ref[s], axis=-1, keepdims=True) + ab4_ref[s]
        out = jnp.where(spf == s, o, out)

    energy_ref[...] = jnp.sum(out.reshape(G, n, 1), axis=1,
                              keepdims=True) + coulomb
    pred_ref[...] = predf


def kernel(species, coordinates, net_charge, params):
    N, n = species.shape
    G = 16
    grid = N // G
    R = G * n

    spf = species.reshape(N * n, 1)
    coordsf = coordinates.reshape(N * n, 3)
    cx = coordinates[:, :, 0]
    cy = coordinates[:, :, 1]
    cz = coordinates[:, :, 2]
    nc3 = net_charge.reshape(N, 1, 1)

    qn = params["q_net"]
    an = params["ani_nets"]
    qw1, qb1 = qn[0][0], qn[0][1].reshape(1, -1)
    qw2, qb2 = qn[1][0], qn[1][1].reshape(1, -1)
    qw3, qb3 = qn[2][0], qn[2][1].reshape(1, -1)
    qw4, qb4 = qn[3][0].reshape(1, -1), qn[3][1].reshape(1, 1)

    aw1 = jnp.stack([net[0][0][:AEV_LEN] for net in an])
    awq = jnp.stack([net[0][0][AEV_LEN:AEV_LEN + 1] for net in an])
    awe = jnp.stack([net[0][0][AEV_LEN + 1:AEV_LEN + 2] for net in an])
    ab1 = jnp.stack([net[0][1].reshape(1, -1) for net in an])
    aw2 = jnp.stack([net[1][0] for net in an])
    ab2 = jnp.stack([net[1][1].reshape(1, -1) for net in an])
    aw3 = jnp.stack([net[2][0] for net in an])
    ab3 = jnp.stack([net[2][1].reshape(1, -1) for net in an])
    aw4 = jnp.stack([net[3][0].reshape(1, -1) for net in an])
    ab4 = jnp.stack([net[3][1].reshape(1, 1) for net in an])

    def rowsblk(i):
        return (i, 0)

    def molblk3(i):
        return (i, 0, 0)

    def full2(i):
        return (0, 0)

    def full3(i):
        return (0, 0, 0)

    in_specs = [
        pl.BlockSpec((R, 1), rowsblk),
        pl.BlockSpec((R, 3), rowsblk),
        pl.BlockSpec((G, n), rowsblk),
        pl.BlockSpec((G, n), rowsblk),
        pl.BlockSpec((G, n), rowsblk),
        pl.BlockSpec((G, 1, 1), molblk3),
        pl.BlockSpec((3, AEV_LEN), full2),
        pl.BlockSpec((1, AEV_LEN), full2),
        pl.BlockSpec(qw1.shape, full2), pl.BlockSpec(qb1.shape, full2),
        pl.BlockSpec(qw2.shape, full2), pl.BlockSpec(qb2.shape, full2),
        pl.BlockSpec(qw3.shape, full2), pl.BlockSpec(qb3.shape, full2),
        pl.BlockSpec(qw4.shape, full2), pl.BlockSpec(qb4.shape, full2),
        pl.BlockSpec(aw1.shape, full3), pl.BlockSpec(awq.shape, full3),
        pl.BlockSpec(awe.shape, full3), pl.BlockSpec(ab1.shape, full3),
        pl.BlockSpec(aw2.shape, full3), pl.BlockSpec(ab2.shape, full3),
        pl.BlockSpec(aw3.shape, full3), pl.BlockSpec(ab3.shape, full3),
        pl.BlockSpec(aw4.shape, full3), pl.BlockSpec(ab4.shape, full3),
    ]
    out_specs = [
        pl.BlockSpec((G, 1, 1), molblk3),
        pl.BlockSpec((R, 1), rowsblk),
    ]
    out_shape = [
        jax.ShapeDtypeStruct((N, 1, 1), jnp.float32),
        jax.ShapeDtypeStruct((N * n, 1), jnp.float32),
    ]

    energy, pred = pl.pallas_call(
        _fused_kernel,
        grid=(grid,),
        in_specs=in_specs,
        out_specs=out_specs,
        out_shape=out_shape,
    )(spf, coordsf, cx, cy, cz, nc3,
      params["W_aev"], params["b_env"].reshape(1, -1),
      qw1, qb1, qw2, qb2, qw3, qb3, qw4, qb4,
      aw1, awq, awe, ab1, aw2, ab2, aw3, ab3, aw4, ab4)

    return species, energy.reshape(N), pred.reshape(N, n)


# speedup vs baseline: 1.3669x; 1.0008x over previous
import jax
import jax.numpy as jnp
from jax import lax
from jax.experimental import pallas as pl
from jax.experimental.pallas import tpu as pltpu

A0 = 0.529177249
RC = 5.2
AEV_LEN = 512
N_SPECIES = 4


def _dot3(x, w):
    xh = x.astype(jnp.bfloat16)
    xl = (x - xh.astype(jnp.float32)).astype(jnp.bfloat16)
    wh = w.astype(jnp.bfloat16)
    wl = (w - wh.astype(jnp.float32)).astype(jnp.bfloat16)
    f32 = jnp.float32
    return (jnp.dot(xh, wh, preferred_element_type=f32)
            + jnp.dot(xh, wl, preferred_element_type=f32)
            + jnp.dot(xl, wh, preferred_element_type=f32))


def _celu(x):
    return jnp.where(x > 0, x, 0.1 * (jnp.exp(jnp.minimum(x, 0.0) / 0.1) - 1.0))


def _fused_kernel(spf_ref, coordsf_ref, cx_ref, cy_ref, cz_ref, nc_ref,
                  wa_ref, benv_ref,
                  qw1_ref, qb1_ref, qw2_ref, qb2_ref, qw3_ref, qb3_ref,
                  qw4_ref, qb4_ref,
                  aw1_ref, awq_ref, awe_ref, ab1_ref,
                  aw2_ref, ab2_ref, aw3_ref, ab3_ref, aw4_ref, ab4_ref,
                  energy_ref, pred_ref):
    G, n = cx_ref.shape
    R = G * n
    coordsf = coordsf_ref[...]
    cxs = coordsf[:, 0:1].reshape(G, n, 1)
    cys = coordsf[:, 1:2].reshape(G, n, 1)
    czs = coordsf[:, 2:3].reshape(G, n, 1)
    cxl = cx_ref[...][:, None, :]
    cyl = cy_ref[...][:, None, :]
    czl = cz_ref[...][:, None, :]

    dx = cxs - cxl
    dy = cys - cyl
    dz = czs - czl
    d2 = dx * dx + dy * dy + dz * dz
    row = lax.broadcasted_iota(jnp.int32, (n, n), 0)
    col = lax.broadcasted_iota(jnp.int32, (n, n), 1)
    offdiag = (row != col).astype(jnp.float32)[None]

    d_aev = jnp.sqrt(d2 + 1e-12)
    fc = 0.5 * jnp.cos(jnp.pi * jnp.minimum(d_aev, RC) / RC) + 0.5
    env = jnp.sum(fc * offdiag, axis=-1, keepdims=True)
    def _b(v):
        return v.astype(jnp.bfloat16).astype(jnp.float32)

    lin = (_b(cxs.reshape(R, 1)) * _b(wa_ref[0:1, :])
           + _b(cys.reshape(R, 1)) * _b(wa_ref[1:2, :])
           + _b(czs.reshape(R, 1)) * _b(wa_ref[2:3, :]))
    aev = jnp.tanh(lin + env.reshape(R, 1) * benv_ref[...])

    f32 = jnp.float32
    h = _celu(jnp.dot(aev, qw1_ref[...], preferred_element_type=f32) + qb1_ref[...])
    h = _celu(jnp.dot(h, qw2_ref[...], preferred_element_type=f32) + qb2_ref[...])
    h = _celu(jnp.dot(h, qw3_ref[...], preferred_element_type=f32) + qb3_ref[...])
    q = jnp.sum(h * qw4_ref[...], axis=-1, keepdims=True) + qb4_ref[...]

    q3 = q.reshape(G, n, 1)
    denom = jnp.maximum(jnp.sum(q3 * q3, axis=1, keepdims=True), 1e-8)
    excess = nc_ref[...] - jnp.sum(q3, axis=1, keepdims=True)
    pred = q3 + excess * (q3 * q3) / denom

    dist = jnp.sqrt(d2 + 1e-16) * (1.0 / A0)
    rinv = offdiag / dist
    esp = lax.dot_general(rinv, pred, (((2,), (1,)), ((0,), (0,))),
                          preferred_element_type=jnp.float32)
    coulomb = 0.5 * jnp.sum(pred * esp, axis=1, keepdims=True)

    predf = pred.reshape(R, 1)
    espf = esp.reshape(R, 1)
    spf = spf_ref[...]
    out = jnp.zeros((R, 1), jnp.float32)
    for s in range(N_SPECIES):
        hs = _celu(jnp.dot(aev, aw1_ref[s], preferred_element_type=jnp.float32)
                   + predf * awq_ref[s] + espf * awe_ref[s] + ab1_ref[s])
        hs = _celu(jnp.dot(hs, aw2_ref[s], preferred_element_type=jnp.float32)
                   + ab2_ref[s])
        hs = _celu(jnp.dot(hs, aw3_ref[s], preferred_element_type=jnp.float32)
                   + ab3_ref[s])
        o = jnp.sum(hs * aw4_ref[s], axis=-1, keepdims=True) + ab4_ref[s]
        out = jnp.where(spf == s, o, out)

    energy_ref[...] = jnp.sum(out.reshape(G, n, 1), axis=1,
                              keepdims=True) + coulomb
    pred_ref[...] = predf


def kernel(species, coordinates, net_charge, params):
    N, n = species.shape
    G = 32
    grid = N // G
    R = G * n

    spf = species.reshape(N * n, 1)
    coordsf = coordinates.reshape(N * n, 3)
    cx = coordinates[:, :, 0]
    cy = coordinates[:, :, 1]
    cz = coordinates[:, :, 2]
    nc3 = net_charge.reshape(N, 1, 1)

    qn = params["q_net"]
    an = params["ani_nets"]
    qw1, qb1 = qn[0][0], qn[0][1].reshape(1, -1)
    qw2, qb2 = qn[1][0], qn[1][1].reshape(1, -1)
    qw3, qb3 = qn[2][0], qn[2][1].reshape(1, -1)
    qw4, qb4 = qn[3][0].reshape(1, -1), qn[3][1].reshape(1, 1)

    aw1 = jnp.stack([net[0][0][:AEV_LEN] for net in an])
    awq = jnp.stack([net[0][0][AEV_LEN:AEV_LEN + 1] for net in an])
    awe = jnp.stack([net[0][0][AEV_LEN + 1:AEV_LEN + 2] for net in an])
    ab1 = jnp.stack([net[0][1].reshape(1, -1) for net in an])
    aw2 = jnp.stack([net[1][0] for net in an])
    ab2 = jnp.stack([net[1][1].reshape(1, -1) for net in an])
    aw3 = jnp.stack([net[2][0] for net in an])
    ab3 = jnp.stack([net[2][1].reshape(1, -1) for net in an])
    aw4 = jnp.stack([net[3][0].reshape(1, -1) for net in an])
    ab4 = jnp.stack([net[3][1].reshape(1, 1) for net in an])

    def rowsblk(i):
        return (i, 0)

    def molblk3(i):
        return (i, 0, 0)

    def full2(i):
        return (0, 0)

    def full3(i):
        return (0, 0, 0)

    in_specs = [
        pl.BlockSpec((R, 1), rowsblk),
        pl.BlockSpec((R, 3), rowsblk),
        pl.BlockSpec((G, n), rowsblk),
        pl.BlockSpec((G, n), rowsblk),
        pl.BlockSpec((G, n), rowsblk),
        pl.BlockSpec((G, 1, 1), molblk3),
        pl.BlockSpec((3, AEV_LEN), full2),
        pl.BlockSpec((1, AEV_LEN), full2),
        pl.BlockSpec(qw1.shape, full2), pl.BlockSpec(qb1.shape, full2),
        pl.BlockSpec(qw2.shape, full2), pl.BlockSpec(qb2.shape, full2),
        pl.BlockSpec(qw3.shape, full2), pl.BlockSpec(qb3.shape, full2),
        pl.BlockSpec(qw4.shape, full2), pl.BlockSpec(qb4.shape, full2),
        pl.BlockSpec(aw1.shape, full3), pl.BlockSpec(awq.shape, full3),
        pl.BlockSpec(awe.shape, full3), pl.BlockSpec(ab1.shape, full3),
        pl.BlockSpec(aw2.shape, full3), pl.BlockSpec(ab2.shape, full3),
        pl.BlockSpec(aw3.shape, full3), pl.BlockSpec(ab3.shape, full3),
        pl.BlockSpec(aw4.shape, full3), pl.BlockSpec(ab4.shape, full3),
    ]
    out_specs = [
        pl.BlockSpec((G, 1, 1), molblk3),
        pl.BlockSpec((R, 1), rowsblk),
    ]
    out_shape = [
        jax.ShapeDtypeStruct((N, 1, 1), jnp.float32),
        jax.ShapeDtypeStruct((N * n, 1), jnp.float32),
    ]

    energy, pred = pl.pallas_call(
        _fused_kernel,
        grid=(grid,),
        in_specs=in_specs,
        out_specs=out_specs,
        out_shape=out_shape,
    )(spf, coordsf, cx, cy, cz, nc3,
      params["W_aev"], params["b_env"].reshape(1, -1),
      qw1, qb1, qw2, qb2, qw3, qb3, qw4, qb4,
      aw1, awq, awe, ab1, aw2, ab2, aw3, ab3, aw4, ab4)

    return species, energy.reshape(N), pred.reshape(N, n)
